# baseline (speedup 1.0000x reference)
import functools
import math

import jax
import jax.numpy as jnp
from jax import lax
from jax.experimental import pallas as pl
from jax.experimental.pallas import tpu as pltpu
from jax.experimental.pallas import tpu_sc as plsc


def _sc_gather_build(N, V, NW=32):
    BW = N // NW
    mesh = plsc.VectorSubcoreMesh(core_axis_name="c", subcore_axis_name="s")

    @functools.partial(
        pl.kernel,
        mesh=mesh,
        out_type=jax.ShapeDtypeStruct((N,), jnp.float32),
        scratch_types=[
            pltpu.VMEM((BW,), jnp.int32),
            pltpu.VMEM((BW,), jnp.int32),
            pltpu.VMEM((BW,), jnp.float32),
            pltpu.SemaphoreType.DMA,
        ],
    )
    def gk(x_hbm, tgt_hbm, out_hbm, tg_v, fidx_v, val_v, sem):
        wid = lax.axis_index("s") * 2 + lax.axis_index("c")
        base = wid * BW
        pltpu.sync_copy(tgt_hbm.at[pl.ds(base, BW)], tg_v)
        for j in range(BW // 16):
            tg = tg_v[pl.ds(j * 16, 16)]
            tid = lax.iota(jnp.int32, 16) + (base + j * 16)
            fidx_v[pl.ds(j * 16, 16)] = tid * V + tg
        pltpu.async_copy(x_hbm.at[fidx_v], val_v, sem).wait()
        pltpu.sync_copy(val_v, out_hbm.at[pl.ds(base, BW)])

    return gk


def _stats_body(x_ref, diff_ref, conf_ref, lse_ref, *, C1, C2, logV):
    Tn, V = x_ref.shape
    NC1 = V // C1
    NC2 = V // C2
    NEG = jnp.float32(-3.0e38)

    M1 = x_ref[:, 0:C1]
    M2 = jnp.full((Tn, C1), NEG, jnp.float32)
    for c in range(1, NC1):
        v = x_ref[:, c * C1:(c + 1) * C1]
        M2 = jnp.maximum(M2, jnp.minimum(M1, v))
        M1 = jnp.maximum(M1, v)
    m = jnp.max(M1, axis=1, keepdims=True)
    eqm = M1 == m
    dupc = jnp.sum(eqm.astype(jnp.float32), axis=1, keepdims=True)
    cand = jnp.where(eqm, M2, M1)
    m2 = jnp.max(cand, axis=1, keepdims=True)
    m2 = jnp.where(dupc > 1.5, m, m2)

    den = jnp.zeros((Tn, 1), jnp.float32)
    s = jnp.zeros((Tn, 1), jnp.float32)
    for c in range(NC2):
        v = x_ref[:, c * C2:(c + 1) * C2]
        z = v - m
        e = jnp.exp(z)
        den = den + jnp.sum(e, axis=1, keepdims=True)
        s = s + jnp.sum(z * e, axis=1, keepdims=True)

    logden = jnp.log(den)
    entropy = logden - s / den
    conf = 1.0 / den
    margin = (1.0 - jnp.exp(m2 - m)) / den
    diff = 0.5 * (entropy / jnp.float32(logV)) + 0.5 * (1.0 - margin)

    diff_ref[0] = diff
    conf_ref[0] = conf
    lse_ref[0] = m + logden


def _shl(x, s):
    n = x.shape[1]
    z = jnp.zeros((1, s), x.dtype)
    return jnp.concatenate([z, x[:, :n - s]], axis=1)


def _select_body(diff_ref, conf_ref, lse_ref, tv_ref, out_ref, *, N):
    d = diff_ref[...]
    cf = conf_ref[...]
    p = lse_ref[...] - tv_ref[...]

    avg = jnp.sum(cf) / jnp.float32(N)
    ratio = jnp.clip(0.25 * (1.0 + 0.5 * (0.5 - avg)), 0.0, 1.0)
    k = jnp.maximum(jnp.int32(1), jnp.floor(ratio * N).astype(jnp.int32))
    kf = k.astype(jnp.float32)

    b = lax.bitcast_convert_type(d, jnp.int32)
    key = jnp.where(b < 0, b ^ jnp.int32(0x7FFFFFFF), b)

    cnt0 = jnp.sum((key >= 0).astype(jnp.int32))
    prefix = jnp.where(cnt0 >= k, jnp.int32(0), jnp.int32(-2**31))
    for bit in range(30, -1, -1):
        cand = prefix | jnp.int32(1 << bit)
        cnt = jnp.sum((key >= cand).astype(jnp.int32))
        prefix = jnp.where(cnt >= k, cand, prefix)
    t = prefix

    gt = key > t
    cnt_gt = jnp.sum(gt.astype(jnp.int32))
    mt = k - cnt_gt

    tie = (key == t).astype(jnp.int32)
    cum = tie
    sft = 1
    while sft < N:
        cum = cum + _shl(cum, sft)
        sft *= 2

    inc = (tie > 0) & (cum <= mt)
    num = jnp.sum(jnp.where(gt, p, 0.0)) + jnp.sum(jnp.where(inc, p, 0.0))
    out_ref[0, 0] = num / kf


def _build(N, V, interpret=False):
    Tn = 64 if N % 64 == 0 else N
    NB = N // Tn
    C1 = 256 if V % 256 == 0 else V
    C2 = 3200 if V % 3200 == 0 else V

    stats = pl.pallas_call(
        functools.partial(_stats_body, C1=C1, C2=C2, logV=math.log(float(V))),
        grid=(NB,),
        in_specs=[pl.BlockSpec((Tn, V), lambda i: (i, 0))],
        out_specs=[
            pl.BlockSpec((1, Tn, 1), lambda i: (i, 0, 0)),
            pl.BlockSpec((1, Tn, 1), lambda i: (i, 0, 0)),
            pl.BlockSpec((1, Tn, 1), lambda i: (i, 0, 0)),
        ],
        out_shape=[
            jax.ShapeDtypeStruct((NB, Tn, 1), jnp.float32),
            jax.ShapeDtypeStruct((NB, Tn, 1), jnp.float32),
            jax.ShapeDtypeStruct((NB, Tn, 1), jnp.float32),
        ],
        interpret=interpret,
    )

    select = pl.pallas_call(
        functools.partial(_select_body, N=N),
        in_specs=[
            pl.BlockSpec(memory_space=pltpu.VMEM),
            pl.BlockSpec(memory_space=pltpu.VMEM),
            pl.BlockSpec(memory_space=pltpu.VMEM),
            pl.BlockSpec(memory_space=pltpu.VMEM),
        ],
        out_specs=pl.BlockSpec(memory_space=pltpu.SMEM),
        out_shape=jax.ShapeDtypeStruct((1, 1), jnp.float32),
        interpret=interpret,
    )
    return stats, select, Tn, NB


def kernel(logits, targets):
    B, S, V = logits.shape
    N = B * S
    stats, select, Tn, NB = _build(N, V)
    x = logits.reshape(N, V)
    tv = _sc_gather_build(N, V)(logits.reshape(N * V), targets.reshape(N))
    diff, conf, lse = stats(x)
    out = select(diff.reshape(1, N), conf.reshape(1, N), lse.reshape(1, N),
                 tv.reshape(1, N))
    return out[0, 0]


# speedup vs baseline: 2.4409x; 2.4409x over previous
import functools
import math

import jax
import jax.numpy as jnp
from jax.experimental import pallas as pl
from jax.experimental.pallas import tpu as pltpu


def _stats_body(x_ref, tgt_ref, diff_ref, conf_ref, pt_ref, *, C1, C2, logV):
    Tn, V = x_ref.shape
    NC1 = V // C1
    NC2 = V // C2
    NEG = jnp.float32(-3.0e38)

    M1 = x_ref[:, 0:C1]
    M2 = jnp.full((Tn, C1), NEG, jnp.float32)
    for c in range(1, NC1):
        v = x_ref[:, c * C1:(c + 1) * C1]
        M2 = jnp.maximum(M2, jnp.minimum(M1, v))
        M1 = jnp.maximum(M1, v)
    m = jnp.max(M1, axis=1, keepdims=True)
    eqm = M1 == m
    dupc = jnp.sum(eqm.astype(jnp.float32), axis=1, keepdims=True)
    cand = jnp.where(eqm, M2, M1)
    m2 = jnp.max(cand, axis=1, keepdims=True)
    m2 = jnp.where(dupc > 1.5, m, m2)

    tgt = tgt_ref[0]

    den = jnp.zeros((Tn, 1), jnp.float32)
    s = jnp.zeros((Tn, 1), jnp.float32)
    tv = jnp.zeros((Tn, 1), jnp.float32)
    for c in range(NC2):
        v = x_ref[:, c * C2:(c + 1) * C2]
        z = v - m
        e = jnp.exp(z)
        den = den + jnp.sum(e, axis=1, keepdims=True)
        s = s + jnp.sum(z * e, axis=1, keepdims=True)
        ii = jax.lax.broadcasted_iota(jnp.int32, (Tn, C2), 1) + (c * C2)
        tv = tv + jnp.sum(jnp.where(ii == tgt, v, 0.0), axis=1, keepdims=True)

    logden = jnp.log(den)
    entropy = logden - s / den
    conf = 1.0 / den
    margin = (1.0 - jnp.exp(m2 - m)) / den
    diff = 0.5 * (entropy / jnp.float32(logV)) + 0.5 * (1.0 - margin)
    pt = m + logden - tv

    diff_ref[0] = diff
    conf_ref[0] = conf
    pt_ref[0] = pt


def _shl(x, s):
    n = x.shape[1]
    z = jnp.zeros((1, s), x.dtype)
    return jnp.concatenate([z, x[:, :n - s]], axis=1)


def _select_body(diff_ref, conf_ref, pt_ref, out_ref, *, N):
    d = diff_ref[...]
    cf = conf_ref[...]
    p = pt_ref[...]

    avg = jnp.sum(cf) / jnp.float32(N)
    ratio = jnp.clip(0.25 * (1.0 + 0.5 * (0.5 - avg)), 0.0, 1.0)
    k = jnp.maximum(jnp.int32(1), jnp.floor(ratio * N).astype(jnp.int32))
    kf = k.astype(jnp.float32)

    b = jax.lax.bitcast_convert_type(d, jnp.int32)
    key = jnp.where(b < 0, b ^ jnp.int32(0x7FFFFFFF), b)

    cnt0 = jnp.sum((key >= 0).astype(jnp.int32))
    prefix = jnp.where(cnt0 >= k, jnp.int32(0), jnp.int32(-2**31))
    for bit in range(30, -1, -1):
        cand = prefix | jnp.int32(1 << bit)
        cnt = jnp.sum((key >= cand).astype(jnp.int32))
        prefix = jnp.where(cnt >= k, cand, prefix)
    t = prefix

    gt = key > t
    cnt_gt = jnp.sum(gt.astype(jnp.int32))
    mt = k - cnt_gt

    tie = (key == t).astype(jnp.int32)
    cum = tie
    sft = 1
    while sft < N:
        cum = cum + _shl(cum, sft)
        sft *= 2

    inc = (tie > 0) & (cum <= mt)
    num = jnp.sum(jnp.where(gt, p, 0.0)) + jnp.sum(jnp.where(inc, p, 0.0))
    out_ref[0, 0] = num / kf


def _build(N, V, interpret=False):
    Tn = 128 if N % 128 == 0 else N
    NB = N // Tn
    C1 = 256 if V % 256 == 0 else V
    C2 = 3200 if V % 3200 == 0 else V

    stats = pl.pallas_call(
        functools.partial(_stats_body, C1=C1, C2=C2, logV=math.log(float(V))),
        grid=(NB,),
        in_specs=[
            pl.BlockSpec((Tn, V), lambda i: (i, 0)),
            pl.BlockSpec((1, Tn, 1), lambda i: (i, 0, 0)),
        ],
        out_specs=[
            pl.BlockSpec((1, Tn, 1), lambda i: (i, 0, 0)),
            pl.BlockSpec((1, Tn, 1), lambda i: (i, 0, 0)),
            pl.BlockSpec((1, Tn, 1), lambda i: (i, 0, 0)),
        ],
        out_shape=[
            jax.ShapeDtypeStruct((NB, Tn, 1), jnp.float32),
            jax.ShapeDtypeStruct((NB, Tn, 1), jnp.float32),
            jax.ShapeDtypeStruct((NB, Tn, 1), jnp.float32),
        ],
        interpret=interpret,
    )

    select = pl.pallas_call(
        functools.partial(_select_body, N=N),
        in_specs=[
            pl.BlockSpec(memory_space=pltpu.VMEM),
            pl.BlockSpec(memory_space=pltpu.VMEM),
            pl.BlockSpec(memory_space=pltpu.VMEM),
        ],
        out_specs=pl.BlockSpec(memory_space=pltpu.SMEM),
        out_shape=jax.ShapeDtypeStruct((1, 1), jnp.float32),
        interpret=interpret,
    )
    return stats, select, Tn, NB


def kernel(logits, targets):
    B, S, V = logits.shape
    N = B * S
    stats, select, Tn, NB = _build(N, V)
    x = logits.reshape(N, V)
    t = targets.reshape(NB, Tn, 1)
    diff, conf, pt = stats(x, t)
    out = select(diff.reshape(1, N), conf.reshape(1, N), pt.reshape(1, N))
    return out[0, 0]
